# baseline (speedup 1.0000x reference)
import functools

import jax
import jax.numpy as jnp
from jax import lax
from jax.experimental import pallas as pl
from jax.experimental.pallas import tpu as pltpu
from jax.experimental.pallas import tpu_sc as plsc

BATCH = 16384
HIST = 50
D = 32
V = 1000000
NW = 32
PR = V // 4

C1_W = 256
C1_NBLK = V // C1_W
C1_TAIL = C1_NBLK * C1_W
C1_IT = -(-C1_NBLK // NW)

BW = BATCH // NW
HALF = 256

_mesh = plsc.VectorSubcoreMesh(core_axis_name="c", subcore_axis_name="s")


@functools.partial(
    pl.kernel,
    mesh=_mesh,
    out_type=jax.ShapeDtypeStruct((PR, 128), jnp.float32),
    compiler_params=pltpu.CompilerParams(use_tc_tiling_on_sc=True, needs_layout_passes=False),
    scratch_types=[
        pltpu.VMEM((2, D, C1_W), jnp.float32),
        pltpu.VMEM((2, C1_W // 4, 128), jnp.float32),
        pltpu.SemaphoreType.DMA((2,)),
        pltpu.SemaphoreType.DMA((2,)),
    ],
)
def _repack(tableT_hbm, tail_hbm, p_hbm, src_v, dst_v, isem, osem):
    wid = lax.axis_index("s") * 2 + lax.axis_index("c")
    lane = lax.iota(jnp.int32, 16)
    rowoff = lane >> 2
    colbase = (lane & 3) * 32

    def col_off(i):
        return pl.multiple_of((wid + NW * i) * C1_W, C1_W)

    def in_copy(i, j):
        return pltpu.make_async_copy(
            tableT_hbm.at[:, pl.ds(col_off(i), C1_W)], src_v.at[j], isem.at[j]
        )

    def out_copy(i, j):
        row = pl.multiple_of(col_off(i) // 4, C1_W // 4)
        return pltpu.make_async_copy(
            dst_v.at[j], p_hbm.at[pl.ds(row, C1_W // 4), :], osem.at[j]
        )

    def valid(i):
        return (wid + NW * i) < C1_NBLK

    for j in (0, 1):

        @pl.when(valid(j))
        def _(j=j):
            in_copy(j, j).start()

    def outer(t, carry):
        for j in (0, 1):
            i = 2 * t + j

            @pl.when(valid(i))
            def _(i=i, j=j):
                @pl.when(i >= 2)
                def _():
                    out_copy(i - 2, j).wait()

                in_copy(i, j).wait()
                rows = [rowoff + 4 * u for u in range(C1_W // 16)]
                for d in range(D):
                    col_d = colbase + d
                    for u in range(C1_W // 16):
                        v = src_v[j, d, pl.ds(16 * u, 16)]
                        plsc.store_scatter(dst_v.at[j], [rows[u], col_d], v)
                out_copy(i, j).start()

                @pl.when(valid(i + 2))
                def _():
                    in_copy(i + 2, j).start()

        return carry

    lax.fori_loop(0, (C1_IT + 1) // 2, outer, 0)
    for j in (0, 1):
        out_copy(j, j).wait()

    @pl.when(wid == 0)
    def _():
        pltpu.sync_copy(tail_hbm, src_v.at[0, :, pl.ds(0, 128)])
        for u in range(4):
            row_u = rowoff + 4 * u
            for d in range(D):
                v = src_v[0, d, pl.ds(16 * u, 16)]
                plsc.store_scatter(dst_v.at[0], [row_u, colbase + d], v)
        pltpu.sync_copy(
            dst_v.at[0, pl.ds(0, 16), :], p_hbm.at[pl.ds(C1_TAIL // 4, 16), :]
        )


@functools.partial(
    pl.kernel,
    mesh=_mesh,
    out_type=jax.ShapeDtypeStruct((HIST, D, BATCH), jnp.float32),
    compiler_params=pltpu.CompilerParams(use_tc_tiling_on_sc=True, needs_layout_passes=False),
    scratch_types=[
        pltpu.VMEM((HIST, BW), jnp.int32),
        pltpu.VMEM((2, 2, 128), jnp.int32),
        pltpu.VMEM((2, HALF), jnp.int32),
        pltpu.VMEM((2, HALF, 128), jnp.float32),
        pltpu.VMEM((D, BW), jnp.float32),
        pltpu.SemaphoreType.DMA((2,)),
        pltpu.SemaphoreType.DMA,
    ],
)
def _gather_t(tok_hbm, p_hbm, out_hbm, idx_all, gidx, qv, rows_v, out_blk, gsem, osem):
    wid = lax.axis_index("s") * 2 + lax.axis_index("c")
    b0 = pl.multiple_of(wid * BW, BW)
    pltpu.sync_copy(tok_hbm.at[wid], idx_all)
    lane = lax.iota(jnp.int32, 16)

    def prep(h, half):
        for g in range(16):
            t = idx_all[h, pl.ds(half * HALF + 16 * g, 16)]
            gidx[half, g // 8, pl.ds((g % 8) * 16, 16)] = t >> 2
            qv[half, pl.ds(16 * g, 16)] = (t & 3) * 32

    def gather_copies(half):
        return [
            pltpu.make_async_copy(
                p_hbm.at[gidx.at[half, p]],
                rows_v.at[half, pl.ds(p * 128, 128)],
                gsem.at[half],
            )
            for p in (0, 1)
        ]

    def start_gather(h, half):
        prep(h, half)
        for c in gather_copies(half):
            c.start()

    def extract(half):
        for g in range(16):
            row_g = lane + 16 * g
            q_g = qv[half, pl.ds(16 * g, 16)]
            for d in range(D):
                v = plsc.load_gather(rows_v.at[half], [row_g, q_g + d])
                out_blk[d, pl.ds(half * HALF + 16 * g, 16)] = v

    def out_copy(h):
        return pltpu.make_async_copy(
            out_blk, out_hbm.at[h, :, pl.ds(b0, BW)], osem
        )

    for half in (0, 1):
        start_gather(0, half)

    def body(h, carry):
        for half in (0, 1):
            for c in gather_copies(half):
                c.wait()
            if half == 0:

                @pl.when(h >= 1)
                def _(h=h):
                    out_copy(h - 1).wait()

            extract(half)

            @pl.when(h + 1 < HIST)
            def _(h=h, half=half):
                start_gather(h + 1, half)

        out_copy(h).start()
        return carry

    lax.fori_loop(0, HIST, body, 0)
    out_copy(HIST - 1).wait()


def kernel(input_tokens, table):
    tableT = jnp.transpose(table)
    tail = jnp.pad(tableT[:, C1_TAIL:], ((0, 0), (0, 64)))
    p = _repack(tableT, tail)
    tok = (
        jnp.transpose(input_tokens)
        .reshape(HIST, NW, BW)
        .transpose(1, 0, 2)
        .astype(jnp.int32)
    )
    outT = _gather_t(tok, p)
    return jnp.transpose(outT, (2, 0, 1))


# speedup vs baseline: 1.4349x; 1.4349x over previous
import functools

import jax
import jax.numpy as jnp
from jax import lax
from jax.experimental import pallas as pl
from jax.experimental.pallas import tpu as pltpu
from jax.experimental.pallas import tpu_sc as plsc

BATCH = 16384
HIST = 50
D = 32
V = 1000000
NW = 32
PR = V // 4

C1_W = 256
C1_NBLK = V // C1_W
C1_TAIL = C1_NBLK * C1_W
C1_IT = -(-C1_NBLK // NW)

BW = BATCH // NW
HALF = 256

_mesh = plsc.VectorSubcoreMesh(core_axis_name="c", subcore_axis_name="s")


@functools.partial(
    pl.kernel,
    mesh=_mesh,
    out_type=jax.ShapeDtypeStruct((PR, 128), jnp.float32),
    compiler_params=pltpu.CompilerParams(use_tc_tiling_on_sc=True, needs_layout_passes=False),
    scratch_types=[
        pltpu.VMEM((2, D, C1_W), jnp.float32),
        pltpu.VMEM((2, C1_W // 4, 128), jnp.float32),
        pltpu.SemaphoreType.DMA((2,)),
        pltpu.SemaphoreType.DMA((2,)),
    ],
)
def _repack(tableT_hbm, tail_hbm, p_hbm, src_v, dst_v, isem, osem):
    wid = lax.axis_index("s") * 2 + lax.axis_index("c")
    lane = lax.iota(jnp.int32, 16)
    rowoff = lane >> 2
    colbase = (lane & 3) * 32

    def col_off(i):
        return pl.multiple_of((wid + NW * i) * C1_W, C1_W)

    def in_copy(i, j):
        return pltpu.make_async_copy(
            tableT_hbm.at[:, pl.ds(col_off(i), C1_W)], src_v.at[j], isem.at[j]
        )

    def out_copy(i, j):
        row = pl.multiple_of(col_off(i) // 4, C1_W // 4)
        return pltpu.make_async_copy(
            dst_v.at[j], p_hbm.at[pl.ds(row, C1_W // 4), :], osem.at[j]
        )

    def valid(i):
        return (wid + NW * i) < C1_NBLK

    for j in (0, 1):

        @pl.when(valid(j))
        def _(j=j):
            in_copy(j, j).start()

    def outer(t, carry):
        for j in (0, 1):
            i = 2 * t + j

            @pl.when(valid(i))
            def _(i=i, j=j):
                @pl.when(i >= 2)
                def _():
                    out_copy(i - 2, j).wait()

                in_copy(i, j).wait()
                @plsc.parallel_loop(0, D, unroll=4)
                def _(d, j=j):
                    col_d = colbase + d
                    for u in range(C1_W // 16):
                        v = src_v[j, d, pl.ds(16 * u, 16)]
                        plsc.store_scatter(
                            dst_v.at[j], [rowoff + 4 * u, col_d], v
                        )

                out_copy(i, j).start()

                @pl.when(valid(i + 2))
                def _():
                    in_copy(i + 2, j).start()

        return carry

    lax.fori_loop(0, (C1_IT + 1) // 2, outer, 0)
    for j in (0, 1):
        out_copy(j, j).wait()

    @pl.when(wid == 0)
    def _():
        pltpu.sync_copy(tail_hbm, src_v.at[0, :, pl.ds(0, 128)])
        for u in range(4):
            row_u = rowoff + 4 * u
            for d in range(D):
                v = src_v[0, d, pl.ds(16 * u, 16)]
                plsc.store_scatter(dst_v.at[0], [row_u, colbase + d], v)
        pltpu.sync_copy(
            dst_v.at[0, pl.ds(0, 16), :], p_hbm.at[pl.ds(C1_TAIL // 4, 16), :]
        )


@functools.partial(
    pl.kernel,
    mesh=_mesh,
    out_type=jax.ShapeDtypeStruct((HIST, D, BATCH), jnp.float32),
    compiler_params=pltpu.CompilerParams(use_tc_tiling_on_sc=True, needs_layout_passes=False),
    scratch_types=[
        pltpu.VMEM((HIST, BW), jnp.int32),
        pltpu.VMEM((2, 2, 128), jnp.int32),
        pltpu.VMEM((2, HALF), jnp.int32),
        pltpu.VMEM((2, HALF, 128), jnp.float32),
        pltpu.VMEM((D, BW), jnp.float32),
        pltpu.SemaphoreType.DMA((2,)),
        pltpu.SemaphoreType.DMA,
    ],
)
def _gather_t(tok_hbm, p_hbm, out_hbm, idx_all, gidx, qv, rows_v, out_blk, gsem, osem):
    wid = lax.axis_index("s") * 2 + lax.axis_index("c")
    b0 = pl.multiple_of(wid * BW, BW)
    pltpu.sync_copy(tok_hbm.at[wid], idx_all)
    lane = lax.iota(jnp.int32, 16)

    def prep(h, half):
        for g in range(16):
            t = idx_all[h, pl.ds(half * HALF + 16 * g, 16)]
            gidx[half, g // 8, pl.ds((g % 8) * 16, 16)] = t >> 2
            qv[half, pl.ds(16 * g, 16)] = (t & 3) * 32

    def gather_copies(half):
        return [
            pltpu.make_async_copy(
                p_hbm.at[gidx.at[half, p]],
                rows_v.at[half, pl.ds(p * 128, 128)],
                gsem.at[half],
            )
            for p in (0, 1)
        ]

    def start_gather(h, half):
        prep(h, half)
        for c in gather_copies(half):
            c.start()

    def extract(half):
        @plsc.parallel_loop(0, 16, unroll=4)
        def _(g, half=half):
            row_g = lane + 16 * g
            q_g = qv[half, pl.ds(16 * g, 16)]
            for d in range(D):
                v = plsc.load_gather(rows_v.at[half], [row_g, q_g + d])
                out_blk[d, pl.ds(half * HALF + 16 * g, 16)] = v

    def out_copy(h):
        return pltpu.make_async_copy(
            out_blk, out_hbm.at[h, :, pl.ds(b0, BW)], osem
        )

    for half in (0, 1):
        start_gather(0, half)

    def body(h, carry):
        for half in (0, 1):
            for c in gather_copies(half):
                c.wait()
            if half == 0:

                @pl.when(h >= 1)
                def _(h=h):
                    out_copy(h - 1).wait()

            extract(half)

            @pl.when(h + 1 < HIST)
            def _(h=h, half=half):
                start_gather(h + 1, half)

        out_copy(h).start()
        return carry

    lax.fori_loop(0, HIST, body, 0)
    out_copy(HIST - 1).wait()


def kernel(input_tokens, table):
    tableT = jnp.transpose(table)
    tail = jnp.pad(tableT[:, C1_TAIL:], ((0, 0), (0, 64)))
    p = _repack(tableT, tail)
    tok = (
        jnp.transpose(input_tokens)
        .reshape(HIST, NW, BW)
        .transpose(1, 0, 2)
        .astype(jnp.int32)
    )
    outT = _gather_t(tok, p)
    return jnp.transpose(outT, (2, 0, 1))
